# baseline (speedup 1.0000x reference)
import functools
import math

import jax
import jax.numpy as jnp
from jax import lax
from jax.experimental import pallas as pl
from jax.experimental.pallas import tpu as pltpu
from jax.experimental.pallas import tpu_sc as plsc

N = 10000
E = 160000
D_IN = 128
HID = 64
DEC = 96
T = 2
NA = 5

NC, NS = 2, 16
NW = NC * NS
CH = 1024
KJ = CH // 128
EP = 163840
BLK = 4096
BLKN = 2000
DG_E, DS_E = 128, 128
DG_D, DS_D = 128, 128


def _mesh():
    return plsc.VectorSubcoreMesh(core_axis_name="c", subcore_axis_name="s",
                                  num_cores=NC, num_subcores=NS)


def _sc_gather(D, B, dtype=jnp.float32):
    per_w = B // NW
    n_ch = per_w // CH

    SB = CH // 4

    @functools.partial(
        pl.kernel, mesh=_mesh(),
        out_type=jax.ShapeDtypeStruct((B, D), dtype),
        scratch_types=[pltpu.VMEM((KJ, 128), jnp.int32),
                       pltpu.VMEM((2, SB, D), dtype),
                       pltpu.SemaphoreType.DMA,
                       pltpu.SemaphoreType.DMA,
                       pltpu.SemaphoreType.DMA])
    def k(tbl, idx2, out, idx_v, rows_v, gsem, wsem0, wsem1):
        wid = lax.axis_index("s") * NC + lax.axis_index("c")
        base0 = wid * per_w
        wsems = (wsem0, wsem1)

        def body(i, carry):
            base = pl.multiple_of(base0 + i * CH, CH)
            pltpu.sync_copy(idx2.at[pl.ds(pl.multiple_of(base // 128, 8), KJ)],
                            idx_v)
            for q in range(4):
                b = q % 2
                dst = out.at[pl.ds(base + q * SB, SB)]
                if q < 2:
                    @pl.when(i > 0)
                    def _():
                        pltpu.make_async_copy(rows_v.at[b], dst, wsems[b]).wait()
                else:
                    pltpu.make_async_copy(rows_v.at[b], dst, wsems[b]).wait()
                cps = [pltpu.async_copy(
                           tbl.at[idx_v.at[q * (KJ // 4) + j]],
                           rows_v.at[b].at[pl.ds(j * 128, 128)], gsem)
                       for j in range(KJ // 4)]
                for c in cps:
                    c.wait()
                pltpu.async_copy(rows_v.at[b], dst, wsems[b])
            return carry

        lax.fori_loop(0, n_ch, body, 0)
        last = pl.multiple_of(base0 + (n_ch - 1) * CH, CH)
        for b in range(2):
            pltpu.make_async_copy(
                rows_v.at[b], out.at[pl.ds(last + (2 + b) * SB, SB)],
                wsems[b]).wait()

    return k


def _sc_scatter(Nn, D, n_t):
    split_t = n_t == NC
    per_sc = EP if split_t else EP // NC
    per_tile = per_sc // NS
    n_ch = per_tile // CH
    n_zc = (Nn + CH - 1) // CH
    n_rounds = 1 if split_t else n_t
    SB = CH // 8

    @functools.partial(
        pl.kernel, mesh=_mesh(),
        out_type=jax.ShapeDtypeStruct(
            ((n_t if split_t else NC * n_t) * Nn, D), jnp.float32),
        scratch_types=[pltpu.VMEM((KJ, 128), jnp.int32),
                       pltpu.VMEM((2, SB, D), jnp.float32),
                       pltpu.VMEM_SHARED((Nn, D), jnp.float32),
                       pltpu.SemaphoreType.DMA,
                       pltpu.SemaphoreType.DMA])
    def k(ev, idx2, zeros, out, idx_v, ev_v, acc, lsem0, lsem1):
        cid = lax.axis_index("c")
        sid = lax.axis_index("s")
        lsems = (lsem0, lsem1)
        for t in range(n_rounds):
            for c in range(n_zc):
                rows = min(CH, Nn - c * CH)

                @pl.when(sid == (c % NS))
                def _():
                    pltpu.sync_copy(zeros.at[pl.ds(0, rows)],
                                    acc.at[pl.ds(c * CH, rows)])
            plsc.subcore_barrier()

            if split_t:
                ebase0 = cid * EP + sid * per_tile
                ibase0 = sid * per_tile // 128
            else:
                ebase0 = t * EP + cid * per_sc + sid * per_tile
                ibase0 = (cid * per_sc + sid * per_tile) // 128

            def esrc(i, q):
                return ev.at[pl.ds(pl.multiple_of(
                    ebase0 + i * CH + q * SB, SB), SB)]

            def body(i, carry):
                pltpu.sync_copy(
                    idx2.at[pl.ds(pl.multiple_of(ibase0 + i * (CH // 128), 8),
                                  KJ)], idx_v)
                pltpu.async_copy(esrc(i, 0), ev_v.at[0], lsems[0])
                for q in range(KJ):
                    b = q % 2
                    if q < KJ - 1:
                        pltpu.async_copy(esrc(i, q + 1), ev_v.at[1 - b],
                                         lsems[1 - b])
                    pltpu.make_async_copy(esrc(i, q), ev_v.at[b],
                                          lsems[b]).wait()
                    pltpu.sync_copy(ev_v.at[b], acc.at[idx_v.at[q]], add=True)
                return carry

            lax.fori_loop(0, n_ch, body, 0)
            plsc.subcore_barrier()

            obase = pl.multiple_of(
                (cid if split_t else cid * n_t + t) * Nn, 8)
            for c in range(n_zc):
                rows = min(CH, Nn - c * CH)

                @pl.when(sid == (c % NS))
                def _():
                    pltpu.sync_copy(acc.at[pl.ds(c * CH, rows)],
                                    out.at[pl.ds(obase + c * CH, rows)])
            plsc.subcore_barrier()

    return k


def _sc_scatter2(Nn, D):
    per_tile = EP // NS
    n_ch = per_tile // CH
    n_zc = (Nn + CH - 1) // CH
    SB = CH // 8

    @functools.partial(
        pl.kernel, mesh=_mesh(),
        out_type=jax.ShapeDtypeStruct((NC * Nn, D), jnp.float32),
        scratch_types=[pltpu.VMEM((KJ, 128), jnp.int32),
                       pltpu.VMEM((2, SB, D), jnp.float32),
                       pltpu.VMEM_SHARED((Nn, D), jnp.float32),
                       pltpu.SemaphoreType.DMA,
                       pltpu.SemaphoreType.DMA])
    def k(ev0, ev1, idx2, zeros, out, idx_v, ev_v, acc, lsem0, lsem1):
        cid = lax.axis_index("c")
        sid = lax.axis_index("s")
        lsems = (lsem0, lsem1)
        for c in range(n_zc):
            rows = min(CH, Nn - c * CH)

            @pl.when(sid == (c % NS))
            def _():
                pltpu.sync_copy(zeros.at[pl.ds(0, rows)],
                                acc.at[pl.ds(c * CH, rows)])
        plsc.subcore_barrier()

        ebase0 = sid * per_tile
        ibase0 = sid * per_tile // 128

        def accumulate(ev):
            def esrc(i, q):
                return ev.at[pl.ds(pl.multiple_of(
                    ebase0 + i * CH + q * SB, SB), SB)]

            def body(i, carry):
                pltpu.sync_copy(
                    idx2.at[pl.ds(pl.multiple_of(ibase0 + i * (CH // 128), 8),
                                  KJ)], idx_v)
                pltpu.async_copy(esrc(i, 0), ev_v.at[0], lsems[0])
                for q in range(KJ):
                    b = q % 2
                    if q < KJ - 1:
                        pltpu.async_copy(esrc(i, q + 1), ev_v.at[1 - b],
                                         lsems[1 - b])
                    pltpu.make_async_copy(esrc(i, q), ev_v.at[b],
                                          lsems[b]).wait()
                    pltpu.sync_copy(ev_v.at[b], acc.at[idx_v.at[q]], add=True)
                return carry

            lax.fori_loop(0, n_ch, body, 0)

        @pl.when(cid == 0)
        def _():
            accumulate(ev0)

        @pl.when(cid == 1)
        def _():
            accumulate(ev1)
        plsc.subcore_barrier()

        obase = pl.multiple_of(cid * Nn, 8)
        for c in range(n_zc):
            rows = min(CH, Nn - c * CH)

            @pl.when(sid == (c % NS))
            def _():
                pltpu.sync_copy(acc.at[pl.ds(c * CH, rows)],
                                out.at[pl.ds(obase + c * CH, rows)])
        plsc.subcore_barrier()

    return k


def _silu(a):
    return a * jax.nn.sigmoid(a)


def _tc_edge(H, Dg, Ds, n_t):
    nb_t = EP // BLK
    grid = (n_t * nb_t,)
    G = 2 * n_t * EP

    def body(gr_ref, gc_ref, ef_ref, w1h, w1c, w1r, w1e, b1, w2, b2,
             wc1, bc1, wc2, bc2, out_ref):
        gr = gr_ref[...]
        gc = gc_ref[...]
        hr = gr[:, 0:H]
        hc = gc[:, 0:H]
        diff = gr[:, H:H + 3] - gc[:, H:H + 3]
        radial = jnp.sum(diff * diff, axis=1, keepdims=True)
        a1 = (jnp.dot(hr, w1h[...], preferred_element_type=jnp.float32)
              + jnp.dot(hc, w1c[...], preferred_element_type=jnp.float32)
              + radial * w1r[...]
              + jnp.dot(ef_ref[...], w1e[...], preferred_element_type=jnp.float32)
              + b1[...])
        m1 = _silu(a1)
        m = _silu(jnp.dot(m1, w2[...], preferred_element_type=jnp.float32) + b2[...])
        c1 = _silu(jnp.dot(m, wc1[...], preferred_element_type=jnp.float32) + bc1[...])
        cw = jnp.sum(c1 * wc2[...], axis=1, keepdims=True) + bc2[...]
        trans = diff * cw
        kidx = pl.program_id(0)
        e_in_t = (kidx % nb_t) * BLK + lax.broadcasted_iota(jnp.int32, (BLK, 1), 0)
        mval = (e_in_t < E).astype(jnp.float32)
        ev = jnp.concatenate(
            [m, trans, jnp.ones((BLK, 1), jnp.float32),
             jnp.zeros((BLK, Ds - H - 4), jnp.float32)], axis=1)
        out_ref[...] = ev * mval

    wspec = lambda s: pl.BlockSpec(s, lambda k: (0, 0))
    return pl.pallas_call(
        body,
        grid=grid,
        in_specs=[
            pl.BlockSpec((BLK, Dg), lambda k: (k, 0)),
            pl.BlockSpec((BLK, Dg), lambda k, _o=n_t * nb_t: (k + _o, 0)),
            pl.BlockSpec((BLK, 4), lambda k, _m=nb_t: (k % _m, 0)),
            wspec((H, H)), wspec((H, H)), wspec((1, H)), wspec((4, H)),
            wspec((1, H)), wspec((H, H)), wspec((1, H)), wspec((H, H)),
            wspec((1, H)), wspec((1, H)), wspec((1, 1)),
        ],
        out_specs=pl.BlockSpec((BLK, Ds), lambda k: (k, 0)),
        out_shape=jax.ShapeDtypeStruct((n_t * EP, Ds), jnp.float32),
    )


def _tc_node(H, Dg, Ds, R, nparts=2):
    grid = (R // BLKN,)

    def body(tbl_ref, vb_ref, *rest):
        ag_refs = rest[:nparts]
        wv, bv, wn1h, wn1m, bn1, wn2, bn2, tbl_out, vb_out = rest[nparts:]
        tb = tbl_ref[...]
        h = tb[:, 0:H]
        x = tb[:, H:H + 3]
        v = vb_ref[...][:, 0:3]
        agg = ag_refs[0][...]
        for r in ag_refs[1:]:
            agg = agg + r[...]
        aggm = agg[:, 0:H]
        aggx = agg[:, H:H + 3] / jnp.maximum(agg[:, H + 3:H + 4], 1.0)
        vel = jnp.sum(h * wv[...], axis=1, keepdims=True) + bv[...]
        v_new = vel * v + aggx
        x_new = x + v_new
        hn = _silu(jnp.dot(h, wn1h[...], preferred_element_type=jnp.float32)
                   + jnp.dot(aggm, wn1m[...], preferred_element_type=jnp.float32)
                   + bn1[...])
        h_new = h + jnp.dot(hn, wn2[...], preferred_element_type=jnp.float32) + bn2[...]
        tbl_out[...] = jnp.concatenate(
            [h_new, x_new, jnp.zeros((BLKN, Dg - H - 3), jnp.float32)], axis=1)
        vb_out[...] = jnp.concatenate(
            [v_new, jnp.zeros((BLKN, 5), jnp.float32)], axis=1)

    wspec = lambda s: pl.BlockSpec(s, lambda k: (0, 0))
    nspec = lambda d: pl.BlockSpec((BLKN, d), lambda k: (k, 0))
    return pl.pallas_call(
        body,
        grid=grid,
        in_specs=[nspec(Dg), nspec(8)] + [nspec(Ds)] * nparts +
                 [wspec((1, H)), wspec((1, 1)), wspec((H, H)), wspec((H, H)),
                  wspec((1, H)), wspec((H, H)), wspec((1, H))],
        out_specs=[nspec(Dg), nspec(8)],
        out_shape=[jax.ShapeDtypeStruct((R, Dg), jnp.float32),
                   jax.ShapeDtypeStruct((R, 8), jnp.float32)],
    )


def _tc_embed():
    grid = (N // BLKN,)

    def body(h_ref, x_ref, w_ref, b_ref, out_ref):
        h0 = jnp.dot(h_ref[...], w_ref[...], preferred_element_type=jnp.float32) + b_ref[...]
        out_ref[...] = jnp.concatenate(
            [h0, x_ref[...], jnp.zeros((BLKN, DG_E - HID - 3), jnp.float32)],
            axis=1)

    return pl.pallas_call(
        body,
        grid=grid,
        in_specs=[pl.BlockSpec((BLKN, D_IN), lambda k: (k, 0)),
                  pl.BlockSpec((BLKN, 3), lambda k: (k, 0)),
                  pl.BlockSpec((D_IN, HID), lambda k: (0, 0)),
                  pl.BlockSpec((1, HID), lambda k: (0, 0))],
        out_specs=pl.BlockSpec((BLKN, DG_E), lambda k: (k, 0)),
        out_shape=jax.ShapeDtypeStruct((N, DG_E), jnp.float32),
    )


def _tc_transition():
    grid = (T * N // BLKN,)
    nb = N // BLKN
    emb = math.log(10000.0) / (16 - 1)

    def body(tbl_ref, tf_ref, out_ref):
        tb = tbl_ref[...]
        freqs = jnp.exp(
            -emb * lax.broadcasted_iota(jnp.int32, (1, 16), 1).astype(jnp.float32))
        args = tf_ref[...] * freqs
        out_ref[...] = jnp.concatenate(
            [tb[:, 0:HID], jnp.sin(args), jnp.cos(args), tb[:, HID:HID + 3],
             jnp.zeros((BLKN, DG_D - DEC - 3), jnp.float32)], axis=1)

    return pl.pallas_call(
        body,
        grid=grid,
        in_specs=[pl.BlockSpec((BLKN, DG_E), lambda k, _m=nb: (k % _m, 0)),
                  pl.BlockSpec((BLKN, 1), lambda k: (k, 0))],
        out_specs=pl.BlockSpec((BLKN, DG_D), lambda k: (k, 0)),
        out_shape=jax.ShapeDtypeStruct((T * N, DG_D), jnp.float32),
    )


def _edge_w(p, H):
    w1 = p["edge1"]["W"]
    return (w1[0:H], w1[H:2 * H], w1[2 * H:2 * H + 1], w1[2 * H + 1:2 * H + 5],
            p["edge1"]["b"][None], p["edge2"]["W"], p["edge2"]["b"][None],
            p["coord1"]["W"], p["coord1"]["b"][None],
            p["coord2"]["W"].T, p["coord2"]["b"].reshape(1, 1))


def _node_w(p, H):
    wn1 = p["node1"]["W"]
    return (p["vel"]["W"].T, p["vel"]["b"].reshape(1, 1),
            wn1[0:H], wn1[H:2 * H], p["node1"]["b"][None],
            p["node2"]["W"], p["node2"]["b"][None])


def _impl(x, h, edge_index, edge_fea, v, loc_mean, timeframes, params):
    del loc_mean
    i32 = jnp.int32
    row = edge_index[0]
    col = edge_index[1]
    pad = EP - E
    row_p = jnp.concatenate([row, jnp.zeros((pad,), i32)])
    col_p = jnp.concatenate([col, jnp.zeros((pad,), i32)])
    enc_idx = jnp.concatenate([row_p, col_p]).reshape(-1, 128)
    dec_idx = [jnp.concatenate([row_p + t * N, col_p + t * N]).reshape(-1, 128)
               for t in range(T)]
    sct_idx = row_p.reshape(-1, 128)
    ef_p = jnp.concatenate([edge_fea, jnp.zeros((pad, 4), jnp.float32)])
    zeros_e = jnp.zeros((CH, DS_E), jnp.float32)
    zeros_d = jnp.zeros((CH, DS_D), jnp.float32)

    gather_e = _sc_gather(DG_E, 2 * EP)
    scatter_e = _sc_scatter(N, DS_E, 1)
    edge_e = _tc_edge(HID, DG_E, DS_E, 1)
    node_e = _tc_node(HID, DG_E, DS_E, N)
    scatter_d = _sc_scatter2(N, DS_D)
    edge_d = _tc_edge(DEC, DG_D, DS_D, 1)
    node_d = _tc_node(DEC, DG_D, DS_D, T * N, nparts=1)

    tbl = _tc_embed()(h, x, params["embedding"]["W"],
                      params["embedding"]["b"][None])
    vb = jnp.concatenate([v, jnp.zeros((N, 5), jnp.float32)], axis=1)
    for i in range(2):
        g = gather_e(tbl, enc_idx)
        ev = edge_e(g, g, ef_p, *_edge_w(params["encode"][i], HID))
        agg = scatter_e(ev, sct_idx, zeros_e)
        tbl, vb = node_e(tbl, vb, agg[0:N], agg[N:2 * N],
                         *_node_w(params["encode"][i], HID))

    tf_nodes = jnp.repeat(timeframes.T, NA, axis=1).reshape(T * N, 1)
    tbl = _tc_transition()(tbl, tf_nodes)
    vb = jnp.tile(vb, (T, 1))
    for i in range(2):
        w = _edge_w(params["decode"][i], DEC)
        g0 = gather_e(tbl, dec_idx[0])
        g1 = gather_e(tbl, dec_idx[1])
        ev0 = edge_d(g0, g0, ef_p, *w)
        ev1 = edge_d(g1, g1, ef_p, *w)
        agg = scatter_d(ev0, ev1, sct_idx, zeros_d)
        tbl, vb = node_d(tbl, vb, agg,
                         *_node_w(params["decode"][i], DEC))

    return tbl[:, DEC:DEC + 3], vb[:, 0:3], tbl[:, 0:DEC]


_run = jax.jit(_impl)


def kernel(x, h, edge_index, edge_fea, v, loc_mean, timeframes, params):
    return _run(x, h, edge_index, edge_fea, v, loc_mean, timeframes, params)


# speedup vs baseline: 1.0251x; 1.0251x over previous
import functools
import math

import jax
import jax.numpy as jnp
from jax import lax
from jax.experimental import pallas as pl
from jax.experimental.pallas import tpu as pltpu
from jax.experimental.pallas import tpu_sc as plsc

N = 10000
E = 160000
D_IN = 128
HID = 64
DEC = 96
T = 2
NA = 5

NC, NS = 2, 16
NW = NC * NS
CH = 1024
KJ = CH // 128
EP = 163840
BLK = 4096
BLKN = 2000
DG_E, DS_E = 128, 128
DG_D, DS_D = 128, 128


def _mesh():
    return plsc.VectorSubcoreMesh(core_axis_name="c", subcore_axis_name="s",
                                  num_cores=NC, num_subcores=NS)


def _sc_gather(D, B, dtype=jnp.float32):
    per_w = B // NW
    n_ch = per_w // CH

    SB = CH // 4

    @functools.partial(
        pl.kernel, mesh=_mesh(),
        out_type=jax.ShapeDtypeStruct((B, D), dtype),
        scratch_types=[pltpu.VMEM((KJ, 128), jnp.int32),
                       pltpu.VMEM((2, SB, D), dtype),
                       pltpu.SemaphoreType.DMA,
                       pltpu.SemaphoreType.DMA,
                       pltpu.SemaphoreType.DMA])
    def k(tbl, idx2, out, idx_v, rows_v, gsem, wsem0, wsem1):
        wid = lax.axis_index("s") * NC + lax.axis_index("c")
        base0 = wid * per_w
        wsems = (wsem0, wsem1)

        def body(i, carry):
            base = pl.multiple_of(base0 + i * CH, CH)
            pltpu.sync_copy(idx2.at[pl.ds(pl.multiple_of(base // 128, 8), KJ)],
                            idx_v)
            for q in range(4):
                b = q % 2
                dst = out.at[pl.ds(base + q * SB, SB)]
                if q < 2:
                    @pl.when(i > 0)
                    def _():
                        pltpu.make_async_copy(rows_v.at[b], dst, wsems[b]).wait()
                else:
                    pltpu.make_async_copy(rows_v.at[b], dst, wsems[b]).wait()
                cps = [pltpu.async_copy(
                           tbl.at[idx_v.at[q * (KJ // 4) + j]],
                           rows_v.at[b].at[pl.ds(j * 128, 128)], gsem)
                       for j in range(KJ // 4)]
                for c in cps:
                    c.wait()
                pltpu.async_copy(rows_v.at[b], dst, wsems[b])
            return carry

        lax.fori_loop(0, n_ch, body, 0)
        last = pl.multiple_of(base0 + (n_ch - 1) * CH, CH)
        for b in range(2):
            pltpu.make_async_copy(
                rows_v.at[b], out.at[pl.ds(last + (2 + b) * SB, SB)],
                wsems[b]).wait()

    return k


def _sc_scatter(Nn, D, n_t):
    split_t = n_t == NC
    per_sc = EP if split_t else EP // NC
    per_tile = per_sc // NS
    n_ch = per_tile // CH
    n_zc = (Nn + CH - 1) // CH
    n_rounds = 1 if split_t else n_t
    SB = CH // 8

    @functools.partial(
        pl.kernel, mesh=_mesh(),
        out_type=jax.ShapeDtypeStruct(
            ((n_t if split_t else NC * n_t) * Nn, D), jnp.float32),
        scratch_types=[pltpu.VMEM((KJ, 128), jnp.int32),
                       pltpu.VMEM((2, SB, D), jnp.float32),
                       pltpu.VMEM_SHARED((Nn, D), jnp.float32),
                       pltpu.SemaphoreType.DMA,
                       pltpu.SemaphoreType.DMA])
    def k(ev, idx2, zeros, out, idx_v, ev_v, acc, lsem0, lsem1):
        cid = lax.axis_index("c")
        sid = lax.axis_index("s")
        lsems = (lsem0, lsem1)
        for t in range(n_rounds):
            for c in range(n_zc):
                rows = min(CH, Nn - c * CH)

                @pl.when(sid == (c % NS))
                def _():
                    pltpu.sync_copy(zeros.at[pl.ds(0, rows)],
                                    acc.at[pl.ds(c * CH, rows)])
            plsc.subcore_barrier()

            if split_t:
                ebase0 = cid * EP + sid * per_tile
                ibase0 = sid * per_tile // 128
            else:
                ebase0 = t * EP + cid * per_sc + sid * per_tile
                ibase0 = (cid * per_sc + sid * per_tile) // 128

            def esrc(i, q):
                return ev.at[pl.ds(pl.multiple_of(
                    ebase0 + i * CH + q * SB, SB), SB)]

            def body(i, carry):
                pltpu.sync_copy(
                    idx2.at[pl.ds(pl.multiple_of(ibase0 + i * (CH // 128), 8),
                                  KJ)], idx_v)
                pltpu.async_copy(esrc(i, 0), ev_v.at[0], lsems[0])
                for q in range(KJ):
                    b = q % 2
                    if q < KJ - 1:
                        pltpu.async_copy(esrc(i, q + 1), ev_v.at[1 - b],
                                         lsems[1 - b])
                    pltpu.make_async_copy(esrc(i, q), ev_v.at[b],
                                          lsems[b]).wait()
                    pltpu.sync_copy(ev_v.at[b], acc.at[idx_v.at[q]], add=True)
                return carry

            lax.fori_loop(0, n_ch, body, 0)
            plsc.subcore_barrier()

            obase = pl.multiple_of(
                (cid if split_t else cid * n_t + t) * Nn, 8)
            for c in range(n_zc):
                rows = min(CH, Nn - c * CH)

                @pl.when(sid == (c % NS))
                def _():
                    pltpu.sync_copy(acc.at[pl.ds(c * CH, rows)],
                                    out.at[pl.ds(obase + c * CH, rows)])
            plsc.subcore_barrier()

    return k


def _silu(a):
    return a * jax.nn.sigmoid(a)


def _tc_edge(H, Dg, Ds, n_t):
    nb_t = EP // BLK
    grid = (n_t * nb_t,)
    G = 2 * n_t * EP

    def body(gr_ref, gc_ref, ef_ref, w1h, w1c, w1r, w1e, b1, w2, b2,
             wc1, bc1, wc2, bc2, out_ref):
        gr = gr_ref[...]
        gc = gc_ref[...]
        hr = gr[:, 0:H]
        hc = gc[:, 0:H]
        diff = gr[:, H:H + 3] - gc[:, H:H + 3]
        radial = jnp.sum(diff * diff, axis=1, keepdims=True)
        a1 = (jnp.dot(hr, w1h[...], preferred_element_type=jnp.float32)
              + jnp.dot(hc, w1c[...], preferred_element_type=jnp.float32)
              + radial * w1r[...]
              + jnp.dot(ef_ref[...], w1e[...], preferred_element_type=jnp.float32)
              + b1[...])
        m1 = _silu(a1)
        m = _silu(jnp.dot(m1, w2[...], preferred_element_type=jnp.float32) + b2[...])
        c1 = _silu(jnp.dot(m, wc1[...], preferred_element_type=jnp.float32) + bc1[...])
        cw = jnp.sum(c1 * wc2[...], axis=1, keepdims=True) + bc2[...]
        trans = diff * cw
        kidx = pl.program_id(0)
        e_in_t = (kidx % nb_t) * BLK + lax.broadcasted_iota(jnp.int32, (BLK, 1), 0)
        mval = (e_in_t < E).astype(jnp.float32)
        ev = jnp.concatenate(
            [m, trans, jnp.ones((BLK, 1), jnp.float32),
             jnp.zeros((BLK, Ds - H - 4), jnp.float32)], axis=1)
        out_ref[...] = ev * mval

    wspec = lambda s: pl.BlockSpec(s, lambda k: (0, 0))
    return pl.pallas_call(
        body,
        grid=grid,
        in_specs=[
            pl.BlockSpec((BLK, Dg), lambda k: (k, 0)),
            pl.BlockSpec((BLK, Dg), lambda k, _o=n_t * nb_t: (k + _o, 0)),
            pl.BlockSpec((BLK, 4), lambda k, _m=nb_t: (k % _m, 0)),
            wspec((H, H)), wspec((H, H)), wspec((1, H)), wspec((4, H)),
            wspec((1, H)), wspec((H, H)), wspec((1, H)), wspec((H, H)),
            wspec((1, H)), wspec((1, H)), wspec((1, 1)),
        ],
        out_specs=pl.BlockSpec((BLK, Ds), lambda k: (k, 0)),
        out_shape=jax.ShapeDtypeStruct((n_t * EP, Ds), jnp.float32),
    )


def _tc_node(H, Dg, Ds, R, nparts=2):
    grid = (R // BLKN,)

    def body(tbl_ref, vb_ref, *rest):
        ag_refs = rest[:nparts]
        wv, bv, wn1h, wn1m, bn1, wn2, bn2, tbl_out, vb_out = rest[nparts:]
        tb = tbl_ref[...]
        h = tb[:, 0:H]
        x = tb[:, H:H + 3]
        v = vb_ref[...][:, 0:3]
        agg = ag_refs[0][...]
        for r in ag_refs[1:]:
            agg = agg + r[...]
        aggm = agg[:, 0:H]
        aggx = agg[:, H:H + 3] / jnp.maximum(agg[:, H + 3:H + 4], 1.0)
        vel = jnp.sum(h * wv[...], axis=1, keepdims=True) + bv[...]
        v_new = vel * v + aggx
        x_new = x + v_new
        hn = _silu(jnp.dot(h, wn1h[...], preferred_element_type=jnp.float32)
                   + jnp.dot(aggm, wn1m[...], preferred_element_type=jnp.float32)
                   + bn1[...])
        h_new = h + jnp.dot(hn, wn2[...], preferred_element_type=jnp.float32) + bn2[...]
        tbl_out[...] = jnp.concatenate(
            [h_new, x_new, jnp.zeros((BLKN, Dg - H - 3), jnp.float32)], axis=1)
        vb_out[...] = jnp.concatenate(
            [v_new, jnp.zeros((BLKN, 5), jnp.float32)], axis=1)

    wspec = lambda s: pl.BlockSpec(s, lambda k: (0, 0))
    nspec = lambda d: pl.BlockSpec((BLKN, d), lambda k: (k, 0))
    return pl.pallas_call(
        body,
        grid=grid,
        in_specs=[nspec(Dg), nspec(8)] + [nspec(Ds)] * nparts +
                 [wspec((1, H)), wspec((1, 1)), wspec((H, H)), wspec((H, H)),
                  wspec((1, H)), wspec((H, H)), wspec((1, H))],
        out_specs=[nspec(Dg), nspec(8)],
        out_shape=[jax.ShapeDtypeStruct((R, Dg), jnp.float32),
                   jax.ShapeDtypeStruct((R, 8), jnp.float32)],
    )


def _tc_embed():
    grid = (N // BLKN,)

    def body(h_ref, x_ref, w_ref, b_ref, out_ref):
        h0 = jnp.dot(h_ref[...], w_ref[...], preferred_element_type=jnp.float32) + b_ref[...]
        out_ref[...] = jnp.concatenate(
            [h0, x_ref[...], jnp.zeros((BLKN, DG_E - HID - 3), jnp.float32)],
            axis=1)

    return pl.pallas_call(
        body,
        grid=grid,
        in_specs=[pl.BlockSpec((BLKN, D_IN), lambda k: (k, 0)),
                  pl.BlockSpec((BLKN, 3), lambda k: (k, 0)),
                  pl.BlockSpec((D_IN, HID), lambda k: (0, 0)),
                  pl.BlockSpec((1, HID), lambda k: (0, 0))],
        out_specs=pl.BlockSpec((BLKN, DG_E), lambda k: (k, 0)),
        out_shape=jax.ShapeDtypeStruct((N, DG_E), jnp.float32),
    )


def _tc_transition():
    grid = (T * N // BLKN,)
    nb = N // BLKN
    emb = math.log(10000.0) / (16 - 1)

    def body(tbl_ref, tf_ref, out_ref):
        tb = tbl_ref[...]
        freqs = jnp.exp(
            -emb * lax.broadcasted_iota(jnp.int32, (1, 16), 1).astype(jnp.float32))
        args = tf_ref[...] * freqs
        out_ref[...] = jnp.concatenate(
            [tb[:, 0:HID], jnp.sin(args), jnp.cos(args), tb[:, HID:HID + 3],
             jnp.zeros((BLKN, DG_D - DEC - 3), jnp.float32)], axis=1)

    return pl.pallas_call(
        body,
        grid=grid,
        in_specs=[pl.BlockSpec((BLKN, DG_E), lambda k, _m=nb: (k % _m, 0)),
                  pl.BlockSpec((BLKN, 1), lambda k: (k, 0))],
        out_specs=pl.BlockSpec((BLKN, DG_D), lambda k: (k, 0)),
        out_shape=jax.ShapeDtypeStruct((T * N, DG_D), jnp.float32),
    )


def _edge_w(p, H):
    w1 = p["edge1"]["W"]
    return (w1[0:H], w1[H:2 * H], w1[2 * H:2 * H + 1], w1[2 * H + 1:2 * H + 5],
            p["edge1"]["b"][None], p["edge2"]["W"], p["edge2"]["b"][None],
            p["coord1"]["W"], p["coord1"]["b"][None],
            p["coord2"]["W"].T, p["coord2"]["b"].reshape(1, 1))


def _node_w(p, H):
    wn1 = p["node1"]["W"]
    return (p["vel"]["W"].T, p["vel"]["b"].reshape(1, 1),
            wn1[0:H], wn1[H:2 * H], p["node1"]["b"][None],
            p["node2"]["W"], p["node2"]["b"][None])


def _impl(x, h, edge_index, edge_fea, v, loc_mean, timeframes, params):
    del loc_mean
    i32 = jnp.int32
    row = edge_index[0]
    col = edge_index[1]
    pad = EP - E
    row_p = jnp.concatenate([row, jnp.zeros((pad,), i32)])
    col_p = jnp.concatenate([col, jnp.zeros((pad,), i32)])
    enc_idx = jnp.concatenate([row_p, col_p]).reshape(-1, 128)
    dec_idx = jnp.concatenate([row_p, row_p + N, col_p, col_p + N]).reshape(-1, 128)
    sct_idx = row_p.reshape(-1, 128)
    ef_p = jnp.concatenate([edge_fea, jnp.zeros((pad, 4), jnp.float32)])
    zeros_e = jnp.zeros((CH, DS_E), jnp.float32)
    zeros_d = jnp.zeros((CH, DS_D), jnp.float32)

    gather_e = _sc_gather(DG_E, 2 * EP)
    scatter_e = _sc_scatter(N, DS_E, 1)
    edge_e = _tc_edge(HID, DG_E, DS_E, 1)
    node_e = _tc_node(HID, DG_E, DS_E, N)
    gather_d = _sc_gather(DG_D, 2 * T * EP)
    scatter_d = _sc_scatter(N, DS_D, T)
    edge_d = _tc_edge(DEC, DG_D, DS_D, T)
    node_d = _tc_node(DEC, DG_D, DS_D, T * N, nparts=1)

    tbl = _tc_embed()(h, x, params["embedding"]["W"],
                      params["embedding"]["b"][None])
    vb = jnp.concatenate([v, jnp.zeros((N, 5), jnp.float32)], axis=1)
    for i in range(2):
        g = gather_e(tbl, enc_idx)
        ev = edge_e(g, g, ef_p, *_edge_w(params["encode"][i], HID))
        agg = scatter_e(ev, sct_idx, zeros_e)
        tbl, vb = node_e(tbl, vb, agg[0:N], agg[N:2 * N],
                         *_node_w(params["encode"][i], HID))

    tf_nodes = jnp.repeat(timeframes.T, NA, axis=1).reshape(T * N, 1)
    tbl = _tc_transition()(tbl, tf_nodes)
    vb = jnp.tile(vb, (T, 1))
    for i in range(2):
        g = gather_d(tbl, dec_idx)
        ev = edge_d(g, g, ef_p, *_edge_w(params["decode"][i], DEC))
        agg = scatter_d(ev, sct_idx, zeros_d)
        tbl, vb = node_d(tbl, vb, agg,
                         *_node_w(params["decode"][i], DEC))

    return tbl[:, DEC:DEC + 3], vb[:, 0:3], tbl[:, 0:DEC]


_run = jax.jit(_impl)


def kernel(x, h, edge_index, edge_fea, v, loc_mean, timeframes, params):
    return _run(x, h, edge_index, edge_fea, v, loc_mean, timeframes, params)
